# Initial kernel scaffold; baseline (speedup 1.0000x reference)
import jax
import jax.numpy as jnp
from jax.experimental import pallas as pl


def kernel(x, W0, b0, ew0_0, eb0_0, ew0_1, eb0_1, nw0_0, nb0_0, nw0_1, nb0_1, nw0_2, nb0_2, ew1_0, eb1_0, ew1_1, eb1_1, nw1_0, nb1_0, nw1_1, nb1_1, nw1_2, nb1_2):
    raise NotImplementedError("write your pallas kernel here")


# speedup vs baseline: 1.9791x; 1.9791x over previous
import jax
import jax.numpy as jnp
from jax.experimental import pallas as pl

B = 64
N = 64
D = 64
OUT = 3
OUT_PAD = 128
ALPHA0 = 0.1
ALPHA1 = 0.1


def _lrelu(v, a):
    return jnp.where(v >= 0, v, a * v)


def _dot(a, b):
    return jnp.dot(a.astype(jnp.bfloat16), b.astype(jnp.bfloat16),
                   preferred_element_type=jnp.float32)


def _bf(v):
    return v.astype(jnp.bfloat16).astype(jnp.float32)


def _proj_kernel(x_ref, w_ref, b_ref, out_ref):
    out_ref[...] = (
        _dot(x_ref[...], w_ref[...])
        + b_ref[...]
    )


def _mp_block(h, Wi, Wj, wd, eb, W1, b1, n0a, n0b, nb0, n1, nb1, n2, nb2,
              alpha, last):
    A = _dot(h, Wi)
    Bm = _dot(h, Wj)
    diff = h[:, None, :] - h[None, :, :]
    dist = jnp.sum(diff * diff, axis=2)
    A2 = A + eb
    pre = (A2[:, None, :] + Bm[None, :, :]
           + _bf(dist)[:, :, None] * _bf(wd)[None, :, :])
    e = _lrelu(pre, alpha)
    e2 = _lrelu(
        _dot(e.reshape(N * N, 64), W1)
        + b1, alpha)
    agg = jnp.sum(e2.reshape(N, N, 64), axis=1)
    n = _lrelu(_dot(h, n0a) + _dot(agg, n0b) + nb0, alpha)
    n = _lrelu(_dot(n, n1) + nb1, alpha)
    n = _dot(n, n2) + nb2
    if not last:
        n = _lrelu(n, alpha)
    return n


def _mp_kernel(h_ref,
               wi0, wj0, wd0, e0b, w01, b01, na0, nb0_, nbi0, n01, nbi01,
               n02, nbi02,
               wi1, wj1, wd1, e1b, w11, b11, na1, nb1_, nbi1, n11, nbi11,
               n12, nbi12,
               out_ref):
    h = h_ref[0]
    h1 = _mp_block(h, wi0[...], wj0[...], wd0[...], e0b[...], w01[...],
                   b01[...], na0[...], nb0_[...], nbi0[...], n01[...],
                   nbi01[...], n02[...], nbi02[...], ALPHA0, False)
    out = _mp_block(h1, wi1[...], wj1[...], wd1[...], e1b[...], w11[...],
                    b11[...], na1[...], nb1_[...], nbi1[...], n11[...],
                    nbi11[...], n12[...], nbi12[...], ALPHA1, True)
    out_ref[0] = out


def kernel(x, W0, b0, ew0_0, eb0_0, ew0_1, eb0_1, nw0_0, nb0_0, nw0_1, nb0_1,
           nw0_2, nb0_2, ew1_0, eb1_0, ew1_1, eb1_1, nw1_0, nb1_0, nw1_1,
           nb1_1, nw1_2, nb1_2):
    H = pl.pallas_call(
        _proj_kernel,
        out_shape=jax.ShapeDtypeStruct((B, N * D), jnp.float32),
    )(x, W0, b0.reshape(1, N * D))
    h0 = H.reshape(B, N, D)

    def split_edge(ew):
        return ew[:D], ew[D:2 * D], ew[2 * D:2 * D + 1]

    wi0, wj0, wd0 = split_edge(ew0_0)
    wi1, wj1, wd1 = split_edge(ew1_0)
    na0, nb0h = nw0_0[:D], nw0_0[D:]
    na1, nb1h = nw1_0[:D], nw1_0[D:]
    n12p = jnp.pad(nw1_2, ((0, 0), (0, OUT_PAD - OUT)))
    nbi12p = jnp.pad(nb1_2, (0, OUT_PAD - OUT)).reshape(1, OUT_PAD)

    row = lambda v: v.reshape(1, -1)
    weights = (
        wi0, wj0, wd0, row(eb0_0), ew0_1, row(eb0_1), na0, nb0h, row(nb0_0),
        nw0_1, row(nb0_1), nw0_2, row(nb0_2),
        wi1, wj1, wd1, row(eb1_0), ew1_1, row(eb1_1), na1, nb1h, row(nb1_0),
        nw1_1, row(nb1_1), n12p, nbi12p,
    )

    wspecs = [pl.BlockSpec(w.shape, lambda b, nd=w.ndim: (0,) * nd)
              for w in weights]

    out = pl.pallas_call(
        _mp_kernel,
        grid=(B,),
        in_specs=[pl.BlockSpec((1, N, D), lambda b: (b, 0, 0))] + wspecs,
        out_specs=pl.BlockSpec((1, N, OUT_PAD), lambda b: (b, 0, 0)),
        out_shape=jax.ShapeDtypeStruct((B, N, OUT_PAD), jnp.float32),
    )(h0, *weights)

    return out[:, :, :OUT]


# speedup vs baseline: 2.0095x; 1.0153x over previous
import jax
import jax.numpy as jnp
from jax.experimental import pallas as pl

B = 64
N = 64
D = 64
OUT = 3
OUT_PAD = 128
ALPHA0 = 0.1
ALPHA1 = 0.1


def _lrelu(v, a):
    return jnp.where(v >= 0, v, a * v)


def _dot(a, b):
    return jnp.dot(a.astype(jnp.bfloat16), b.astype(jnp.bfloat16),
                   preferred_element_type=jnp.float32)


def _bf(v):
    return v.astype(jnp.bfloat16).astype(jnp.float32)


def _proj_kernel(x_ref, w_ref, b_ref, out_ref):
    out_ref[...] = (
        _dot(x_ref[...], w_ref[...])
        + b_ref[...]
    )


def _mp_block(h, Wi, Wj, wd, eb, W1, b1, n0a, n0b, nb0, n1, nb1, n2, nb2,
              alpha, last):
    A = _dot(h, Wi)
    Bm = _dot(h, Wj)
    hh = h * h
    G = jax.lax.dot_general(h, h, (((1,), (1,)), ((), ())),
                            preferred_element_type=jnp.float32,
                            precision=jax.lax.Precision.HIGHEST)
    r = jnp.sum(hh, axis=1, keepdims=True)
    ones_row = jnp.ones((1, N), jnp.float32)
    rT = jax.lax.dot_general(ones_row, hh, (((1,), (1,)), ((), ())),
                             preferred_element_type=jnp.float32,
                             precision=jax.lax.Precision.HIGHEST)
    dist = r + rT - 2.0 * G
    A2 = A + eb
    pre = (A2[:, None, :] + Bm[None, :, :]
           + _bf(dist)[:, :, None] * _bf(wd)[None, :, :])
    e = _lrelu(pre, alpha)
    e2 = _lrelu(
        _dot(e.reshape(N * N, 64), W1)
        + b1, alpha)
    agg = jnp.sum(e2.reshape(N, N, 64), axis=1)
    n = _lrelu(_dot(h, n0a) + _dot(agg, n0b) + nb0, alpha)
    n = _lrelu(_dot(n, n1) + nb1, alpha)
    n = _dot(n, n2) + nb2
    if not last:
        n = _lrelu(n, alpha)
    return n


def _mp_kernel(h_ref,
               wi0, wj0, wd0, e0b, w01, b01, na0, nb0_, nbi0, n01, nbi01,
               n02, nbi02,
               wi1, wj1, wd1, e1b, w11, b11, na1, nb1_, nbi1, n11, nbi11,
               n12, nbi12,
               out_ref):
    h = h_ref[0]
    h1 = _mp_block(h, wi0[...], wj0[...], wd0[...], e0b[...], w01[...],
                   b01[...], na0[...], nb0_[...], nbi0[...], n01[...],
                   nbi01[...], n02[...], nbi02[...], ALPHA0, False)
    out = _mp_block(h1, wi1[...], wj1[...], wd1[...], e1b[...], w11[...],
                    b11[...], na1[...], nb1_[...], nbi1[...], n11[...],
                    nbi11[...], n12[...], nbi12[...], ALPHA1, True)
    out_ref[0] = out


def kernel(x, W0, b0, ew0_0, eb0_0, ew0_1, eb0_1, nw0_0, nb0_0, nw0_1, nb0_1,
           nw0_2, nb0_2, ew1_0, eb1_0, ew1_1, eb1_1, nw1_0, nb1_0, nw1_1,
           nb1_1, nw1_2, nb1_2):
    H = pl.pallas_call(
        _proj_kernel,
        out_shape=jax.ShapeDtypeStruct((B, N * D), jnp.float32),
    )(x, W0, b0.reshape(1, N * D))
    h0 = H.reshape(B, N, D)

    def split_edge(ew):
        return ew[:D], ew[D:2 * D], ew[2 * D:2 * D + 1]

    wi0, wj0, wd0 = split_edge(ew0_0)
    wi1, wj1, wd1 = split_edge(ew1_0)
    na0, nb0h = nw0_0[:D], nw0_0[D:]
    na1, nb1h = nw1_0[:D], nw1_0[D:]
    n12p = jnp.pad(nw1_2, ((0, 0), (0, OUT_PAD - OUT)))
    nbi12p = jnp.pad(nb1_2, (0, OUT_PAD - OUT)).reshape(1, OUT_PAD)

    row = lambda v: v.reshape(1, -1)
    weights = (
        wi0, wj0, wd0, row(eb0_0), ew0_1, row(eb0_1), na0, nb0h, row(nb0_0),
        nw0_1, row(nb0_1), nw0_2, row(nb0_2),
        wi1, wj1, wd1, row(eb1_0), ew1_1, row(eb1_1), na1, nb1h, row(nb1_0),
        nw1_1, row(nb1_1), n12p, nbi12p,
    )

    wspecs = [pl.BlockSpec(w.shape, lambda b, nd=w.ndim: (0,) * nd)
              for w in weights]

    out = pl.pallas_call(
        _mp_kernel,
        grid=(B,),
        in_specs=[pl.BlockSpec((1, N, D), lambda b: (b, 0, 0))] + wspecs,
        out_specs=pl.BlockSpec((1, N, OUT_PAD), lambda b: (b, 0, 0)),
        out_shape=jax.ShapeDtypeStruct((B, N, OUT_PAD), jnp.float32),
    )(h0, *weights)

    return out[:, :, :OUT]
